# baseline (speedup 1.0000x reference)
import functools

import jax
import jax.numpy as jnp
from jax.experimental import pallas as pl
from jax.experimental.pallas import tpu as pltpu

_NBUF = 3


def _copy(adj_hbm, bufs, sems, step, block_rows):
    return pltpu.make_async_copy(
        adj_hbm.at[pl.ds(step * block_rows, block_rows), :],
        bufs.at[step % _NBUF],
        sems.at[step % _NBUF],
    )


def _body(block_rows, adj_hbm, x_ref, w_ref, out_ref, xw_ref, bufs, sems):
    i = pl.program_id(0)
    nsteps = pl.num_programs(0)

    @pl.when(i == 0)
    def _():
        xw_ref[:] = jnp.dot(x_ref[:], w_ref[:],
                            preferred_element_type=jnp.float32)
        for b in range(_NBUF - 1):
            _copy(adj_hbm, bufs, sems, b, block_rows).start()

    nxt = i + _NBUF - 1

    @pl.when(nxt < nsteps)
    def _():
        _copy(adj_hbm, bufs, sems, nxt, block_rows).start()

    _copy(adj_hbm, bufs, sems, i, block_rows).wait()

    acc = jnp.dot(bufs[i % _NBUF].astype(jnp.bfloat16),
                  xw_ref[:].astype(jnp.bfloat16),
                  preferred_element_type=jnp.float32)
    out_ref[:] = jnp.maximum(acc, 0.0)


@functools.partial(jax.jit, static_argnames=("block_rows", "interpret"))
def _gcn(adj, inputs, W, block_rows, interpret=False):
    n, _ = adj.shape
    d_out = W.shape[1]
    grid = (n // block_rows,)
    return pl.pallas_call(
        functools.partial(_body, block_rows),
        grid=grid,
        in_specs=[
            pl.BlockSpec(memory_space=pl.ANY),
            pl.BlockSpec(inputs.shape, lambda i: (0, 0)),
            pl.BlockSpec(W.shape, lambda i: (0, 0)),
        ],
        out_specs=pl.BlockSpec((block_rows, d_out), lambda i: (i, 0)),
        out_shape=jax.ShapeDtypeStruct((n, d_out), jnp.float32),
        scratch_shapes=[
            pltpu.VMEM((inputs.shape[0], d_out), jnp.float32),
            pltpu.VMEM((_NBUF, block_rows, n), jnp.float32),
            pltpu.SemaphoreType.DMA((_NBUF,)),
        ],
        interpret=interpret,
    )(adj, inputs, W)


def kernel(adj, inputs, W):
    return _gcn(adj, inputs, W, 400)


# speedup vs baseline: 1.0249x; 1.0249x over previous
import functools

import jax
import jax.numpy as jnp
from jax.experimental import pallas as pl
from jax.experimental.pallas import tpu as pltpu


def _body(adj_ref, x_ref, w_ref, out_ref, xw_ref):
    @pl.when(pl.program_id(0) == 0)
    def _():
        xw_ref[:] = jnp.dot(x_ref[:], w_ref[:],
                            preferred_element_type=jnp.float32)

    acc = jnp.dot(adj_ref[:].astype(jnp.bfloat16),
                  xw_ref[:].astype(jnp.bfloat16),
                  preferred_element_type=jnp.float32)
    out_ref[:] = jnp.maximum(acc, 0.0)


@functools.partial(jax.jit, static_argnames=("block_rows", "interpret"))
def _gcn(adj, inputs, W, block_rows, interpret=False):
    n, _ = adj.shape
    d_out = W.shape[1]
    grid = (n // block_rows,)
    return pl.pallas_call(
        _body,
        grid=grid,
        in_specs=[
            pl.BlockSpec((block_rows, n), lambda i: (i, 0)),
            pl.BlockSpec(inputs.shape, lambda i: (0, 0)),
            pl.BlockSpec(W.shape, lambda i: (0, 0)),
        ],
        out_specs=pl.BlockSpec((block_rows, d_out), lambda i: (i, 0)),
        out_shape=jax.ShapeDtypeStruct((n, d_out), jnp.float32),
        scratch_shapes=[pltpu.VMEM((inputs.shape[0], d_out), jnp.float32)],
        interpret=interpret,
    )(adj, inputs, W)


def kernel(adj, inputs, W):
    n = adj.shape[0]
    block_rows = 8
    for b in range(8, 512, 8):
        if n % b == 0:
            block_rows = b
    return _gcn(adj, inputs, W, block_rows)


# speedup vs baseline: 1.0347x; 1.0095x over previous
import functools

import jax
import jax.numpy as jnp
from jax.experimental import pallas as pl
from jax.experimental.pallas import tpu as pltpu


def _body(adj_ref, x_ref, w_ref, out_ref, xw_ref):
    @pl.when(pl.program_id(0) == 0)
    def _():
        xw_ref[:] = jnp.dot(x_ref[:], w_ref[:],
                            preferred_element_type=jnp.float32)

    acc = jnp.dot(adj_ref[:].astype(jnp.bfloat16),
                  xw_ref[:].astype(jnp.bfloat16),
                  preferred_element_type=jnp.float32)
    out_ref[:] = jnp.maximum(acc, 0.0)


@functools.partial(jax.jit, static_argnames=("block_rows",))
def _gcn(adj, inputs, W, block_rows):
    n, _ = adj.shape
    d_out = W.shape[1]
    grid = (n // block_rows,)
    return pl.pallas_call(
        _body,
        grid=grid,
        in_specs=[
            pl.BlockSpec((block_rows, n), lambda i: (i, 0)),
            pl.BlockSpec(inputs.shape, lambda i: (0, 0)),
            pl.BlockSpec(W.shape, lambda i: (0, 0)),
        ],
        out_specs=pl.BlockSpec((block_rows, d_out), lambda i: (i, 0)),
        out_shape=jax.ShapeDtypeStruct((n, d_out), jnp.float32),
        scratch_shapes=[pltpu.VMEM((inputs.shape[0], d_out), jnp.float32)],
    )(adj, inputs, W)


def kernel(adj, inputs, W):
    n = adj.shape[0]
    block_rows = 8
    for b in range(8, 512, 8):
        if n % b == 0:
            block_rows = b
    return _gcn(adj, inputs, W, block_rows)
